# baseline (speedup 1.0000x reference)
import jax
import jax.numpy as jnp
from jax import lax
from jax.experimental import pallas as pl
from jax.experimental.pallas import tpu as pltpu
from jax.experimental.pallas import tpu_sc as plsc

VOCAB = 1000000
EMB = 64
HID = 256
B = 4096
L = 200

NC = 2
NS = 16
NW = NC * NS
BPW = B // NW
NLANE = 16
NGRP = EMB // NLANE
NBUF = 1
CA = 104
CB = L - CA
PAIRW = 2 * EMB


def _sc_body(mA_hbm, mB_hbm, par_hbm, table_hbm, out_hbm,
             mA_v, mB_v, gA0, gB0, pv0, acc_v, sm0):
    wid = lax.axis_index("s") * NC + lax.axis_index("c")
    pltpu.sync_copy(mA_hbm.at[wid], mA_v)
    pltpu.sync_copy(mB_hbm.at[wid], mB_v)

    gAs = (gA0,)
    gBs = (gB0,)
    pvs = (pv0,)
    sems = (sm0,)

    def fire(r, b):
        pltpu.async_copy(table_hbm.at[mA_v.at[r]], gAs[b], sems[b])
        pltpu.async_copy(table_hbm.at[mB_v.at[r]], gBs[b], sems[b])
        pltpu.async_copy(par_hbm.at[wid, r], pvs[b], sems[b])

    def drain(r, b):
        pltpu.make_async_copy(
            table_hbm.at[mA_v.at[r]], gAs[b], sems[b]).wait()
        pltpu.make_async_copy(
            table_hbm.at[mB_v.at[r]], gBs[b], sems[b]).wait()
        pltpu.make_async_copy(par_hbm.at[wid, r], pvs[b], sems[b]).wait()

    for b in range(NBUF):
        fire(b, b)

    zero = jnp.zeros((NLANE,), jnp.float32)

    def outer(g, carry):
        for b in range(NBUF):
            r = NBUF * g + b
            drain(r, b)

            def redA(i, accs):
                p = pvs[b][i, pl.ds(0, NLANE)]
                out = list(accs)
                for k in range(NGRP):
                    lo = gAs[b][i, pl.ds(NLANE * k, NLANE)]
                    hi = gAs[b][i, pl.ds(EMB + NLANE * k, NLANE)]
                    out[k] = accs[k] + (lo + p * (hi - lo))
                return tuple(out)

            def redB(i, accs):
                p = pvs[b][CA + i, pl.ds(0, NLANE)]
                out = list(accs)
                for k in range(NGRP):
                    lo = gBs[b][i, pl.ds(NLANE * k, NLANE)]
                    hi = gBs[b][i, pl.ds(EMB + NLANE * k, NLANE)]
                    out[k] = accs[k] + (lo + p * (hi - lo))
                return tuple(out)

            accA = lax.fori_loop(0, CA, redA, (zero,) * NGRP, unroll=4)
            accB = lax.fori_loop(0, CB, redB, (zero,) * NGRP, unroll=4)
            for k in range(NGRP):
                acc_v[r, pl.ds(NLANE * k, NLANE)] = accA[k] + accB[k]

            rn = r + NBUF

            @pl.when(rn < BPW)
            def _():
                fire(rn, b)
        return carry

    lax.fori_loop(0, BPW // NBUF, outer, 0)
    pltpu.sync_copy(acc_v, out_hbm.at[pl.ds(wid * BPW, BPW)])


_sc_lookup = pl.kernel(
    _sc_body,
    out_type=jax.ShapeDtypeStruct((B, EMB), jnp.float32),
    mesh=plsc.VectorSubcoreMesh(core_axis_name="c", subcore_axis_name="s"),
    scratch_types=[
        pltpu.VMEM((BPW, CA), jnp.int32),
        pltpu.VMEM((BPW, CB), jnp.int32),
        pltpu.VMEM((CA, PAIRW), jnp.float32),
        pltpu.VMEM((CB, PAIRW), jnp.float32),
        pltpu.VMEM((L, NLANE), jnp.float32),
        pltpu.VMEM((BPW, EMB), jnp.float32),
        pltpu.SemaphoreType.DMA,
    ],
)


def _mlp_body(x_ref, w1_ref, b1_ref, w2_ref, b2_ref, o_ref):
    x = x_ref[...] * (1.0 / L)
    h = jnp.dot(x, w1_ref[...], preferred_element_type=jnp.float32) + b1_ref[...]
    h = jnp.maximum(h, 0.0)
    y = jnp.dot(h, w2_ref[...], preferred_element_type=jnp.float32) + b2_ref[...]
    o_ref[...] = 1.0 / (1.0 + jnp.exp(-y))


def kernel(input_ids, emb_table, W1, b1, W2, b2):
    ids = input_ids.astype(jnp.int32).reshape(NW, BPW, L)
    pairs = ids >> 1
    par = jnp.broadcast_to(
        (ids & 1).astype(jnp.float32)[..., None], (NW, BPW, L, NLANE))
    sums = _sc_lookup(pairs[:, :, :CA], pairs[:, :, CA:], par,
                      emb_table.reshape(VOCAB // 2, PAIRW))
    return pl.pallas_call(
        _mlp_body,
        out_shape=jax.ShapeDtypeStruct((B, 1), jnp.float32),
    )(sums, W1, b1.reshape(1, HID), W2, b2.reshape(1, 1))


# speedup vs baseline: 2.7416x; 2.7416x over previous
import jax
import jax.numpy as jnp
from jax import lax
from jax.experimental import pallas as pl
from jax.experimental.pallas import tpu as pltpu
from jax.experimental.pallas import tpu_sc as plsc

VOCAB = 1000000
EMB = 64
HID = 256
B = 4096
L = 200

NC = 2
NS = 16
NW = NC * NS
BPW = B // NW
NLANE = 16
NGRP = EMB // NLANE
CA = 104
CB = L - CA


def _sc_body(mA_hbm, mB_hbm, table_hbm, out_hbm,
             mA_v, mB_v, gA, gB, acc_v, sm0):
    wid = lax.axis_index("s") * NC + lax.axis_index("c")
    pltpu.sync_copy(mA_hbm.at[wid], mA_v)
    pltpu.sync_copy(mB_hbm.at[wid], mB_v)

    def fire(r):
        pltpu.async_copy(table_hbm.at[mA_v.at[r]], gA, sm0)
        pltpu.async_copy(table_hbm.at[mB_v.at[r]], gB, sm0)

    def drain(r):
        pltpu.make_async_copy(table_hbm.at[mA_v.at[r]], gA, sm0).wait()
        pltpu.make_async_copy(table_hbm.at[mB_v.at[r]], gB, sm0).wait()

    fire(0)

    zero = jnp.zeros((NLANE,), jnp.float32)

    def outer(r, carry):
        drain(r)

        def redA(i, accs):
            return tuple(accs[k] + gA[i, 0, pl.ds(NLANE * k, NLANE)]
                         for k in range(NGRP))

        def redB(i, accs):
            return tuple(accs[k] + gB[i, 0, pl.ds(NLANE * k, NLANE)]
                         for k in range(NGRP))

        accA = lax.fori_loop(0, CA, redA, (zero,) * NGRP, unroll=4)
        accB = lax.fori_loop(0, CB, redB, (zero,) * NGRP, unroll=4)
        for k in range(NGRP):
            acc_v[r, pl.ds(NLANE * k, NLANE)] = accA[k] + accB[k]

        @pl.when(r + 1 < BPW)
        def _():
            fire(r + 1)

        return carry

    lax.fori_loop(0, BPW, outer, 0)
    pltpu.sync_copy(acc_v, out_hbm.at[pl.ds(wid * BPW, BPW)])


_sc_lookup = pl.kernel(
    _sc_body,
    out_type=jax.ShapeDtypeStruct((B, EMB), jnp.float32),
    mesh=plsc.VectorSubcoreMesh(core_axis_name="c", subcore_axis_name="s"),
    scratch_types=[
        pltpu.VMEM((BPW, CA), jnp.int32),
        pltpu.VMEM((BPW, CB), jnp.int32),
        pltpu.VMEM((CA, 1, EMB), jnp.float32),
        pltpu.VMEM((CB, 1, EMB), jnp.float32),
        pltpu.VMEM((BPW, EMB), jnp.float32),
        pltpu.SemaphoreType.DMA,
    ],
)


def _mlp_body(x_ref, w1_ref, b1_ref, w2_ref, b2_ref, o_ref):
    x = x_ref[...] * (1.0 / L)
    h = jnp.dot(x, w1_ref[...], preferred_element_type=jnp.float32) + b1_ref[...]
    h = jnp.maximum(h, 0.0)
    y = jnp.dot(h, w2_ref[...], preferred_element_type=jnp.float32) + b2_ref[...]
    o_ref[...] = 1.0 / (1.0 + jnp.exp(-y))


def kernel(input_ids, emb_table, W1, b1, W2, b2):
    ids = input_ids.astype(jnp.int32).reshape(NW, BPW, L)
    sums = _sc_lookup(ids[:, :, :CA], ids[:, :, CA:],
                      emb_table.reshape(VOCAB, 1, EMB))
    return pl.pallas_call(
        _mlp_body,
        out_shape=jax.ShapeDtypeStruct((B, 1), jnp.float32),
    )(sums, W1, b1.reshape(1, HID), W2, b2.reshape(1, 1))
